# Initial kernel scaffold; baseline (speedup 1.0000x reference)
import jax
import jax.numpy as jnp
from jax.experimental import pallas as pl


def kernel(Z, rbf, idx_i, idx_j, embeddings, W_rbf, b_rbf, W_dense, b_dense):
    raise NotImplementedError("write your pallas kernel here")


# speedup vs baseline: 16.3873x; 16.3873x over previous
import functools

import jax
import jax.numpy as jnp
from jax import lax
from jax.experimental import pallas as pl
from jax.experimental.pallas import tpu as pltpu
from jax.experimental.pallas import tpu_sc as plsc

_NC = 2
_NS = 16
_NW = _NC * _NS

_EDGE_BLOCK = 2560


def _sc_gather(Z, idx_i, idx_j, embeddings):
    NE = idx_i.shape[0]
    NN = Z.shape[0]
    D = embeddings.shape[1]
    e_per = NE // _NW
    n_chunk = 320
    n_sub = 80
    mesh = plsc.VectorSubcoreMesh(core_axis_name="c", subcore_axis_name="s")

    @functools.partial(
        pl.kernel,
        mesh=mesh,
        out_type=[
            jax.ShapeDtypeStruct((NE,), jnp.int32),
            jax.ShapeDtypeStruct((NE,), jnp.int32),
            jax.ShapeDtypeStruct((NN, D), jnp.float32),
        ],
        scratch_types=[
            pltpu.VMEM((NN,), jnp.int32),
            pltpu.VMEM((e_per,), jnp.int32),
            pltpu.VMEM((e_per,), jnp.int32),
            pltpu.VMEM((n_chunk,), jnp.int32),
            pltpu.VMEM((n_chunk, D), jnp.float32),
            pltpu.SemaphoreType.DMA,
        ],
        compiler_params=pltpu.CompilerParams(needs_layout_passes=False),
    )
    def k(z_hbm, ii_hbm, jj_hbm, emb_hbm, zi_hbm, zj_hbm, x0_hbm,
          z_v, ii_v, jj_v, zc_v, rows_v, sem):
        wid = lax.axis_index("s") * _NC + lax.axis_index("c")
        ebase = wid * e_per
        pltpu.sync_copy(z_hbm, z_v)
        pltpu.sync_copy(ii_hbm.at[pl.ds(ebase, e_per)], ii_v)
        pltpu.sync_copy(jj_hbm.at[pl.ds(ebase, e_per)], jj_v)

        def body(t, carry):
            s = pl.ds(t * 16, 16)
            ii_v[s] = plsc.load_gather(z_v, [ii_v[s]])
            jj_v[s] = plsc.load_gather(z_v, [jj_v[s]])
            return carry

        lax.fori_loop(0, e_per // 16, body, 0)
        pltpu.sync_copy(ii_v, zi_hbm.at[pl.ds(ebase, e_per)])
        pltpu.sync_copy(jj_v, zj_hbm.at[pl.ds(ebase, e_per)])

        nbase = jnp.minimum(wid * n_chunk, NN - n_chunk)
        pltpu.sync_copy(z_hbm.at[pl.ds(nbase, n_chunk)], zc_v)
        for t in range(n_chunk // n_sub):
            pltpu.async_copy(
                emb_hbm.at[zc_v.at[pl.ds(t * n_sub, n_sub)]],
                rows_v.at[pl.ds(t * n_sub, n_sub)],
                sem,
            ).wait()
        pltpu.sync_copy(rows_v, x0_hbm.at[pl.ds(nbase, n_chunk)])

    return k(Z, idx_i, idx_j, embeddings)


def _tc_body(zi_ref, zj_ref, rbf_ref, emb_ref, wd_ref, wr_ref, br_ref, bd_ref,
             out_ref, tab_ref, wc_ref, bc_ref):
    B = rbf_ref.shape[0]

    @pl.when(pl.program_id(0) == 0)
    def _():
        emb = emb_ref[...]
        wd = wd_ref[...]
        tab_ref[0:128, :] = jnp.dot(emb, wd[0:128, :],
                                    preferred_element_type=jnp.float32)
        tab_ref[128:256, :] = jnp.dot(emb, wd[128:256, :],
                                      preferred_element_type=jnp.float32)
        wd3 = wd[256:384, :]
        wc_ref[...] = jnp.dot(wr_ref[...], wd3,
                              preferred_element_type=jnp.float32)
        bc_ref[...] = jnp.dot(br_ref[...], wd3,
                              preferred_element_type=jnp.float32) + bd_ref[...]

    zi = zi_ref[...].reshape(1, B)
    zj = zj_ref[...].reshape(1, B)
    iot = lax.broadcasted_iota(jnp.int32, (256, B), 0)
    two_hot = jnp.where((iot == zi) | (iot == zj + 128), 1.0, 0.0)
    g = lax.dot_general(two_hot, tab_ref[...], (((0,), (0,)), ((), ())),
                        preferred_element_type=jnp.float32)
    p = jnp.dot(rbf_ref[...], wc_ref[...],
                preferred_element_type=jnp.float32)
    out_ref[...] = g + p + bc_ref[...]


def _tc_fused(zi, zj, rbf, emb_pad, W_dense, W_rbf, b_rbf, b_dense):
    NE, n_rbf = rbf.shape
    B = _EDGE_BLOCK
    NB = NE // B
    zi3 = zi.reshape(NB, 1, B)
    zj3 = zj.reshape(NB, 1, B)
    return pl.pallas_call(
        _tc_body,
        grid=(NB,),
        in_specs=[
            pl.BlockSpec((1, 1, B), lambda i: (i, 0, 0)),
            pl.BlockSpec((1, 1, B), lambda i: (i, 0, 0)),
            pl.BlockSpec((B, n_rbf), lambda i: (i, 0)),
            pl.BlockSpec((128, 128), lambda i: (0, 0)),
            pl.BlockSpec((384, 128), lambda i: (0, 0)),
            pl.BlockSpec((n_rbf, 128), lambda i: (0, 0)),
            pl.BlockSpec((1, 128), lambda i: (0, 0)),
            pl.BlockSpec((1, 128), lambda i: (0, 0)),
        ],
        out_specs=pl.BlockSpec((B, 128), lambda i: (i, 0)),
        out_shape=jax.ShapeDtypeStruct((NE, 128), jnp.float32),
        scratch_shapes=[
            pltpu.VMEM((256, 128), jnp.float32),
            pltpu.VMEM((n_rbf, 128), jnp.float32),
            pltpu.VMEM((1, 128), jnp.float32),
        ],
        compiler_params=pltpu.CompilerParams(
            dimension_semantics=("arbitrary",),
        ),
    )(zi3, zj3, rbf, emb_pad, W_dense, W_rbf, b_rbf, b_dense)


def kernel(Z, rbf, idx_i, idx_j, embeddings, W_rbf, b_rbf, W_dense, b_dense):
    Z = Z.astype(jnp.int32)
    idx_i = idx_i.astype(jnp.int32)
    idx_j = idx_j.astype(jnp.int32)
    zi, zj, x0 = _sc_gather(Z, idx_i, idx_j, embeddings)
    emb_pad = jnp.pad(embeddings, ((0, 128 - embeddings.shape[0]), (0, 0)))
    x = _tc_fused(zi, zj, rbf, emb_pad, W_dense, W_rbf,
                  b_rbf.reshape(1, -1), b_dense.reshape(1, -1))
    return (x, x0)


# speedup vs baseline: 16.7251x; 1.0206x over previous
import functools

import jax
import jax.numpy as jnp
from jax import lax
from jax.experimental import pallas as pl
from jax.experimental.pallas import tpu as pltpu
from jax.experimental.pallas import tpu_sc as plsc

_NC = 2
_NS = 16
_NW = _NC * _NS

_EDGE_BLOCK = 2560


def _sc_gather(Z, idx_i, idx_j, embeddings):
    NE = idx_i.shape[0]
    NN = Z.shape[0]
    D = embeddings.shape[1]
    e_per = NE // _NW
    n_chunk = 320
    n_sub = 80
    mesh = plsc.VectorSubcoreMesh(core_axis_name="c", subcore_axis_name="s")

    @functools.partial(
        pl.kernel,
        mesh=mesh,
        out_type=[
            jax.ShapeDtypeStruct((NE,), jnp.int32),
            jax.ShapeDtypeStruct((NE,), jnp.int32),
            jax.ShapeDtypeStruct((NN, D), jnp.float32),
        ],
        scratch_types=[
            pltpu.VMEM((NN,), jnp.int32),
            pltpu.VMEM((e_per,), jnp.int32),
            pltpu.VMEM((e_per,), jnp.int32),
            pltpu.VMEM((n_chunk,), jnp.int32),
            pltpu.VMEM((n_chunk, D), jnp.float32),
            pltpu.SemaphoreType.DMA,
        ],
        compiler_params=pltpu.CompilerParams(needs_layout_passes=False),
    )
    def k(z_hbm, ii_hbm, jj_hbm, emb_hbm, zi_hbm, zj_hbm, x0_hbm,
          z_v, ii_v, jj_v, zc_v, rows_v, sem):
        wid = lax.axis_index("s") * _NC + lax.axis_index("c")
        ebase = wid * e_per
        pltpu.sync_copy(z_hbm, z_v)
        pltpu.sync_copy(ii_hbm.at[pl.ds(ebase, e_per)], ii_v)
        pltpu.sync_copy(jj_hbm.at[pl.ds(ebase, e_per)], jj_v)

        def body(t, carry):
            s = pl.ds(t * 16, 16)
            ii_v[s] = plsc.load_gather(z_v, [ii_v[s]])
            jj_v[s] = plsc.load_gather(z_v, [jj_v[s]])
            return carry

        lax.fori_loop(0, e_per // 16, body, 0)
        pltpu.sync_copy(ii_v, zi_hbm.at[pl.ds(ebase, e_per)])
        pltpu.sync_copy(jj_v, zj_hbm.at[pl.ds(ebase, e_per)])

        nbase = jnp.minimum(wid * n_chunk, NN - n_chunk)
        pltpu.sync_copy(z_hbm.at[pl.ds(nbase, n_chunk)], zc_v)
        for t in range(n_chunk // n_sub):
            pltpu.async_copy(
                emb_hbm.at[zc_v.at[pl.ds(t * n_sub, n_sub)]],
                rows_v.at[pl.ds(t * n_sub, n_sub)],
                sem,
            ).wait()
        pltpu.sync_copy(rows_v, x0_hbm.at[pl.ds(nbase, n_chunk)])

    return k(Z, idx_i, idx_j, embeddings)


def _tc_body(zi_ref, zj_ref, rbf_ref, emb_ref, wd_ref, wr_ref, br_ref, bd_ref,
             out_ref, tab_ref, wc_ref, bc_ref):
    B = rbf_ref.shape[0]

    @pl.when(pl.program_id(0) == 0)
    def _():
        emb = emb_ref[...]
        wd = wd_ref[...]
        tab_ref[0:128, :] = jnp.dot(
            emb, wd[0:128, :],
            preferred_element_type=jnp.float32).astype(jnp.bfloat16)
        tab_ref[128:256, :] = jnp.dot(
            emb, wd[128:256, :],
            preferred_element_type=jnp.float32).astype(jnp.bfloat16)
        wd3 = wd[256:384, :]
        wc_ref[...] = jnp.dot(wr_ref[...], wd3,
                              preferred_element_type=jnp.float32)
        bc_ref[...] = jnp.dot(br_ref[...], wd3,
                              preferred_element_type=jnp.float32) + bd_ref[...]

    zi = zi_ref[...].reshape(1, B)
    zj = zj_ref[...].reshape(1, B)
    zib = zi.astype(jnp.int16)
    zjb = zj.astype(jnp.int16) + jnp.int16(128)
    iot = lax.broadcasted_iota(jnp.int16, (256, B), 0)
    ones = jnp.ones((256, B), jnp.bfloat16)
    zeros = jnp.zeros((256, B), jnp.bfloat16)
    two_hot = jnp.where((iot == zib) | (iot == zjb), ones, zeros)
    g = lax.dot_general(two_hot, tab_ref[...], (((0,), (0,)), ((), ())),
                        preferred_element_type=jnp.float32)
    p = jnp.dot(rbf_ref[...], wc_ref[...],
                preferred_element_type=jnp.float32)
    out_ref[...] = g + p + bc_ref[...]


def _tc_fused(zi, zj, rbf, emb_pad, W_dense, W_rbf, b_rbf, b_dense):
    NE, n_rbf = rbf.shape
    B = _EDGE_BLOCK
    NB = NE // B
    zi3 = zi.reshape(NB, 1, B)
    zj3 = zj.reshape(NB, 1, B)
    return pl.pallas_call(
        _tc_body,
        grid=(NB,),
        in_specs=[
            pl.BlockSpec((1, 1, B), lambda i: (i, 0, 0)),
            pl.BlockSpec((1, 1, B), lambda i: (i, 0, 0)),
            pl.BlockSpec((B, n_rbf), lambda i: (i, 0)),
            pl.BlockSpec((128, 128), lambda i: (0, 0)),
            pl.BlockSpec((384, 128), lambda i: (0, 0)),
            pl.BlockSpec((n_rbf, 128), lambda i: (0, 0)),
            pl.BlockSpec((1, 128), lambda i: (0, 0)),
            pl.BlockSpec((1, 128), lambda i: (0, 0)),
        ],
        out_specs=pl.BlockSpec((B, 128), lambda i: (i, 0)),
        out_shape=jax.ShapeDtypeStruct((NE, 128), jnp.float32),
        scratch_shapes=[
            pltpu.VMEM((256, 128), jnp.bfloat16),
            pltpu.VMEM((n_rbf, 128), jnp.float32),
            pltpu.VMEM((1, 128), jnp.float32),
        ],
        compiler_params=pltpu.CompilerParams(
            dimension_semantics=("arbitrary",),
        ),
    )(zi3, zj3, rbf, emb_pad, W_dense, W_rbf, b_rbf, b_dense)


def kernel(Z, rbf, idx_i, idx_j, embeddings, W_rbf, b_rbf, W_dense, b_dense):
    Z = Z.astype(jnp.int32)
    idx_i = idx_i.astype(jnp.int32)
    idx_j = idx_j.astype(jnp.int32)
    zi, zj, x0 = _sc_gather(Z, idx_i, idx_j, embeddings)
    emb_pad = jnp.pad(embeddings, ((0, 128 - embeddings.shape[0]), (0, 0)))
    x = _tc_fused(zi, zj, rbf, emb_pad, W_dense, W_rbf,
                  b_rbf.reshape(1, -1), b_dense.reshape(1, -1))
    return (x, x0)


# speedup vs baseline: 24.5501x; 1.4679x over previous
import functools

import jax
import jax.numpy as jnp
from jax import lax
from jax.experimental import pallas as pl
from jax.experimental.pallas import tpu as pltpu
from jax.experimental.pallas import tpu_sc as plsc

_NC = 2
_NS = 16
_NW = _NC * _NS

_EDGE_BLOCK = 2560


def _sc_gather(Z, idx_i, idx_j, embeddings):
    NE = idx_i.shape[0]
    NN = Z.shape[0]
    D = embeddings.shape[1]
    e_per = NE // _NW
    n_chunk = 320
    n_sub = 80
    mesh = plsc.VectorSubcoreMesh(core_axis_name="c", subcore_axis_name="s")

    @functools.partial(
        pl.kernel,
        mesh=mesh,
        out_type=[
            jax.ShapeDtypeStruct((NE,), jnp.int32),
            jax.ShapeDtypeStruct((NE,), jnp.int32),
            jax.ShapeDtypeStruct((NN, D), jnp.float32),
        ],
        scratch_types=[
            pltpu.VMEM((NN,), jnp.int32),
            pltpu.VMEM((e_per,), jnp.int32),
            pltpu.VMEM((e_per,), jnp.int32),
            pltpu.VMEM((n_chunk,), jnp.int32),
            pltpu.VMEM((n_chunk, D), jnp.float32),
            pltpu.SemaphoreType.DMA,
        ],
        compiler_params=pltpu.CompilerParams(needs_layout_passes=False),
    )
    def k(z_hbm, ii_hbm, jj_hbm, emb_hbm, zi_hbm, zj_hbm, x0_hbm,
          z_v, ii_v, jj_v, zc_v, rows_v, sem):
        wid = lax.axis_index("s") * _NC + lax.axis_index("c")
        ebase = wid * e_per
        pltpu.sync_copy(z_hbm, z_v)
        pltpu.sync_copy(ii_hbm.at[pl.ds(ebase, e_per)], ii_v)
        pltpu.sync_copy(jj_hbm.at[pl.ds(ebase, e_per)], jj_v)

        def body(t, carry):
            s = pl.ds(t * 16, 16)
            ii_v[s] = plsc.load_gather(z_v, [ii_v[s]])
            jj_v[s] = plsc.load_gather(z_v, [jj_v[s]])
            return carry

        lax.fori_loop(0, e_per // 16, body, 0)
        pltpu.sync_copy(ii_v, zi_hbm.at[pl.ds(ebase, e_per)])
        pltpu.sync_copy(jj_v, zj_hbm.at[pl.ds(ebase, e_per)])

        nbase = jnp.minimum(wid * n_chunk, NN - n_chunk)
        pltpu.sync_copy(z_hbm.at[pl.ds(nbase, n_chunk)], zc_v)
        for t in range(n_chunk // n_sub):
            pltpu.async_copy(
                emb_hbm.at[zc_v.at[pl.ds(t * n_sub, n_sub)]],
                rows_v.at[pl.ds(t * n_sub, n_sub)],
                sem,
            ).wait()
        pltpu.sync_copy(rows_v, x0_hbm.at[pl.ds(nbase, n_chunk)])

    return k(Z, idx_i, idx_j, embeddings)


def _tc_body(zi_ref, zj_ref, rbf_ref, emb_ref, wd_ref, wr_ref, br_ref, bd_ref,
             out_ref, tab_ref, wc_ref, bc_ref):
    B = rbf_ref.shape[1]

    @pl.when(pl.program_id(0) == 0)
    def _():
        emb = emb_ref[...]
        wd = wd_ref[...]
        tab_ref[0:128, :] = jnp.dot(
            emb, wd[0:128, :],
            preferred_element_type=jnp.float32).astype(jnp.bfloat16)
        tab_ref[128:256, :] = jnp.dot(
            emb, wd[128:256, :],
            preferred_element_type=jnp.float32).astype(jnp.bfloat16)
        wd3 = wd[256:384, :]
        wc_ref[...] = jnp.dot(wr_ref[...], wd3,
                              preferred_element_type=jnp.float32)
        bc_ref[...] = jnp.dot(br_ref[...], wd3,
                              preferred_element_type=jnp.float32) + bd_ref[...]

    zi = zi_ref[...].reshape(1, B)
    zj = zj_ref[...].reshape(1, B)
    zib = zi.astype(jnp.int16)
    zjb = zj.astype(jnp.int16) + jnp.int16(128)
    iot = lax.broadcasted_iota(jnp.int16, (256, B), 0)
    ones = jnp.ones((256, B), jnp.bfloat16)
    zeros = jnp.zeros((256, B), jnp.bfloat16)
    two_hot = jnp.where((iot == zib) | (iot == zjb), ones, zeros)
    g = lax.dot_general(two_hot, tab_ref[...], (((0,), (0,)), ((), ())),
                        preferred_element_type=jnp.float32)
    p = lax.dot_general(rbf_ref[...], wc_ref[...], (((0,), (0,)), ((), ())),
                        preferred_element_type=jnp.float32)
    out_ref[...] = g + p + bc_ref[...]


def _tc_fused(zi, zj, rbf_t, emb_pad, W_dense, W_rbf, b_rbf, b_dense):
    n_rbf, NE = rbf_t.shape
    B = _EDGE_BLOCK
    NB = NE // B
    zi3 = zi.reshape(NB, 1, B)
    zj3 = zj.reshape(NB, 1, B)
    return pl.pallas_call(
        _tc_body,
        grid=(NB,),
        in_specs=[
            pl.BlockSpec((1, 1, B), lambda i: (i, 0, 0)),
            pl.BlockSpec((1, 1, B), lambda i: (i, 0, 0)),
            pl.BlockSpec((n_rbf, B), lambda i: (0, i)),
            pl.BlockSpec((128, 128), lambda i: (0, 0)),
            pl.BlockSpec((384, 128), lambda i: (0, 0)),
            pl.BlockSpec((n_rbf, 128), lambda i: (0, 0)),
            pl.BlockSpec((1, 128), lambda i: (0, 0)),
            pl.BlockSpec((1, 128), lambda i: (0, 0)),
        ],
        out_specs=pl.BlockSpec((B, 128), lambda i: (i, 0)),
        out_shape=jax.ShapeDtypeStruct((NE, 128), jnp.float32),
        scratch_shapes=[
            pltpu.VMEM((256, 128), jnp.bfloat16),
            pltpu.VMEM((n_rbf, 128), jnp.float32),
            pltpu.VMEM((1, 128), jnp.float32),
        ],
        compiler_params=pltpu.CompilerParams(
            dimension_semantics=("arbitrary",),
        ),
    )(zi3, zj3, rbf_t, emb_pad, W_dense, W_rbf, b_rbf, b_dense)


def kernel(Z, rbf, idx_i, idx_j, embeddings, W_rbf, b_rbf, W_dense, b_dense):
    Z = Z.astype(jnp.int32)
    idx_i = idx_i.astype(jnp.int32)
    idx_j = idx_j.astype(jnp.int32)
    zi, zj, x0 = _sc_gather(Z, idx_i, idx_j, embeddings)
    emb_pad = jnp.pad(embeddings, ((0, 128 - embeddings.shape[0]), (0, 0)))
    x = _tc_fused(zi, zj, rbf.T, emb_pad, W_dense, W_rbf,
                  b_rbf.reshape(1, -1), b_dense.reshape(1, -1))
    return (x, x0)


# speedup vs baseline: 24.7739x; 1.0091x over previous
import functools

import jax
import jax.numpy as jnp
from jax import lax
from jax.experimental import pallas as pl
from jax.experimental.pallas import tpu as pltpu
from jax.experimental.pallas import tpu_sc as plsc

_NC = 2
_NS = 16
_NW = _NC * _NS

_EDGE_BLOCK = 2560


def _sc_gather(Z, idx_i, idx_j, embeddings):
    NE = idx_i.shape[0]
    NN = Z.shape[0]
    D = embeddings.shape[1]
    e_per = NE // _NW
    n_chunk = 320
    n_sub = 80
    mesh = plsc.VectorSubcoreMesh(core_axis_name="c", subcore_axis_name="s")

    @functools.partial(
        pl.kernel,
        mesh=mesh,
        out_type=[
            jax.ShapeDtypeStruct((NE,), jnp.int32),
            jax.ShapeDtypeStruct((NE,), jnp.int32),
            jax.ShapeDtypeStruct((NN, D), jnp.float32),
        ],
        scratch_types=[
            pltpu.VMEM((NN,), jnp.int32),
            pltpu.VMEM((e_per,), jnp.int32),
            pltpu.VMEM((e_per,), jnp.int32),
            pltpu.VMEM((e_per,), jnp.int32),
            pltpu.VMEM((e_per,), jnp.int32),
            pltpu.VMEM((n_chunk,), jnp.int32),
            pltpu.VMEM((n_chunk, D), jnp.float32),
            pltpu.SemaphoreType.DMA,
        ],
        compiler_params=pltpu.CompilerParams(needs_layout_passes=False),
    )
    def k(z_hbm, ii_hbm, jj_hbm, emb_hbm, zi_hbm, zj_hbm, x0_hbm,
          z_v, ii_v, jj_v, zi_v, zj_v, zc_v, rows_v, sem):
        wid = lax.axis_index("s") * _NC + lax.axis_index("c")
        ebase = wid * e_per
        pltpu.sync_copy(z_hbm, z_v)
        pltpu.sync_copy(ii_hbm.at[pl.ds(ebase, e_per)], ii_v)
        pltpu.sync_copy(jj_hbm.at[pl.ds(ebase, e_per)], jj_v)

        @plsc.parallel_loop(0, e_per, step=16, unroll=8)
        def _(t):
            s = pl.ds(t, 16)
            zi_v[s] = plsc.load_gather(z_v, [ii_v[s]])
            zj_v[s] = plsc.load_gather(z_v, [jj_v[s]])

        pltpu.sync_copy(zi_v, zi_hbm.at[pl.ds(ebase, e_per)])
        pltpu.sync_copy(zj_v, zj_hbm.at[pl.ds(ebase, e_per)])

        nbase = jnp.minimum(wid * n_chunk, NN - n_chunk)
        pltpu.sync_copy(z_hbm.at[pl.ds(nbase, n_chunk)], zc_v)
        for t in range(n_chunk // n_sub):
            pltpu.async_copy(
                emb_hbm.at[zc_v.at[pl.ds(t * n_sub, n_sub)]],
                rows_v.at[pl.ds(t * n_sub, n_sub)],
                sem,
            ).wait()
        pltpu.sync_copy(rows_v, x0_hbm.at[pl.ds(nbase, n_chunk)])

    return k(Z, idx_i, idx_j, embeddings)


def _tc_body(zi_ref, zj_ref, rbf_ref, emb_ref, wd_ref, wr_ref, br_ref, bd_ref,
             out_ref, tab_ref, wc_ref, bc_ref):
    B = rbf_ref.shape[1]

    @pl.when(pl.program_id(0) == 0)
    def _():
        emb = emb_ref[...]
        wd = wd_ref[...]
        tab_ref[0:128, :] = jnp.dot(
            emb, wd[0:128, :],
            preferred_element_type=jnp.float32).astype(jnp.bfloat16)
        tab_ref[128:256, :] = jnp.dot(
            emb, wd[128:256, :],
            preferred_element_type=jnp.float32).astype(jnp.bfloat16)
        wd3 = wd[256:384, :]
        wc_ref[...] = jnp.dot(wr_ref[...], wd3,
                              preferred_element_type=jnp.float32)
        bc_ref[...] = jnp.dot(br_ref[...], wd3,
                              preferred_element_type=jnp.float32) + bd_ref[...]

    zi = zi_ref[...].reshape(1, B)
    zj = zj_ref[...].reshape(1, B)
    zib = zi.astype(jnp.int16)
    zjb = zj.astype(jnp.int16) + jnp.int16(128)
    iot = lax.broadcasted_iota(jnp.int16, (256, B), 0)
    ones = jnp.ones((256, B), jnp.bfloat16)
    zeros = jnp.zeros((256, B), jnp.bfloat16)
    two_hot = jnp.where((iot == zib) | (iot == zjb), ones, zeros)
    g = lax.dot_general(two_hot, tab_ref[...], (((0,), (0,)), ((), ())),
                        preferred_element_type=jnp.float32)
    p = lax.dot_general(rbf_ref[...], wc_ref[...], (((0,), (0,)), ((), ())),
                        preferred_element_type=jnp.float32)
    out_ref[...] = g + p + bc_ref[...]


def _tc_fused(zi, zj, rbf_t, emb_pad, W_dense, W_rbf, b_rbf, b_dense):
    n_rbf, NE = rbf_t.shape
    B = _EDGE_BLOCK
    NB = NE // B
    zi3 = zi.reshape(NB, 1, B)
    zj3 = zj.reshape(NB, 1, B)
    return pl.pallas_call(
        _tc_body,
        grid=(NB,),
        in_specs=[
            pl.BlockSpec((1, 1, B), lambda i: (i, 0, 0)),
            pl.BlockSpec((1, 1, B), lambda i: (i, 0, 0)),
            pl.BlockSpec((n_rbf, B), lambda i: (0, i)),
            pl.BlockSpec((128, 128), lambda i: (0, 0)),
            pl.BlockSpec((384, 128), lambda i: (0, 0)),
            pl.BlockSpec((n_rbf, 128), lambda i: (0, 0)),
            pl.BlockSpec((1, 128), lambda i: (0, 0)),
            pl.BlockSpec((1, 128), lambda i: (0, 0)),
        ],
        out_specs=pl.BlockSpec((B, 128), lambda i: (i, 0)),
        out_shape=jax.ShapeDtypeStruct((NE, 128), jnp.float32),
        scratch_shapes=[
            pltpu.VMEM((256, 128), jnp.bfloat16),
            pltpu.VMEM((n_rbf, 128), jnp.float32),
            pltpu.VMEM((1, 128), jnp.float32),
        ],
        compiler_params=pltpu.CompilerParams(
            dimension_semantics=("arbitrary",),
            fuse_transposed_lhs_in_matmul=True,
        ),
    )(zi3, zj3, rbf_t, emb_pad, W_dense, W_rbf, b_rbf, b_dense)


def kernel(Z, rbf, idx_i, idx_j, embeddings, W_rbf, b_rbf, W_dense, b_dense):
    Z = Z.astype(jnp.int32)
    idx_i = idx_i.astype(jnp.int32)
    idx_j = idx_j.astype(jnp.int32)
    zi, zj, x0 = _sc_gather(Z, idx_i, idx_j, embeddings)
    emb_pad = jnp.pad(embeddings, ((0, 128 - embeddings.shape[0]), (0, 0)))
    x = _tc_fused(zi, zj, rbf.T, emb_pad, W_dense, W_rbf,
                  b_rbf.reshape(1, -1), b_dense.reshape(1, -1))
    return (x, x0)


# speedup vs baseline: 26.1747x; 1.0565x over previous
import functools

import jax
import jax.numpy as jnp
from jax import lax
from jax.experimental import pallas as pl
from jax.experimental.pallas import tpu as pltpu
from jax.experimental.pallas import tpu_sc as plsc

_NC = 2
_NS = 16
_NW = _NC * _NS

_EDGE_BLOCK = 2560


_SC_MESH = plsc.VectorSubcoreMesh(core_axis_name="c", subcore_axis_name="s")


def _sc_translate(Z, idx_i, idx_j):
    NE = idx_i.shape[0]
    NN = Z.shape[0]
    e_per = NE // _NW

    @functools.partial(
        pl.kernel,
        mesh=_SC_MESH,
        out_type=[
            jax.ShapeDtypeStruct((NE,), jnp.int32),
            jax.ShapeDtypeStruct((NE,), jnp.int32),
        ],
        scratch_types=[
            pltpu.VMEM((NN,), jnp.int32),
            pltpu.VMEM((e_per,), jnp.int32),
            pltpu.VMEM((e_per,), jnp.int32),
            pltpu.VMEM((e_per,), jnp.int32),
            pltpu.VMEM((e_per,), jnp.int32),
        ],
        compiler_params=pltpu.CompilerParams(needs_layout_passes=False),
    )
    def k(z_hbm, ii_hbm, jj_hbm, zi_hbm, zj_hbm,
          z_v, ii_v, jj_v, zi_v, zj_v):
        wid = lax.axis_index("s") * _NC + lax.axis_index("c")
        ebase = wid * e_per
        pltpu.sync_copy(z_hbm, z_v)
        pltpu.sync_copy(ii_hbm.at[pl.ds(ebase, e_per)], ii_v)
        pltpu.sync_copy(jj_hbm.at[pl.ds(ebase, e_per)], jj_v)

        @plsc.parallel_loop(0, e_per, step=16, unroll=8)
        def _(t):
            s = pl.ds(t, 16)
            zi_v[s] = plsc.load_gather(z_v, [ii_v[s]])
            zj_v[s] = plsc.load_gather(z_v, [jj_v[s]])

        pltpu.sync_copy(zi_v, zi_hbm.at[pl.ds(ebase, e_per)])
        pltpu.sync_copy(zj_v, zj_hbm.at[pl.ds(ebase, e_per)])

    return k(Z, idx_i, idx_j)


def _sc_x0(Z, embeddings):
    NN = Z.shape[0]
    D = embeddings.shape[1]
    n_chunk = 320
    n_sub = 80

    @functools.partial(
        pl.kernel,
        mesh=_SC_MESH,
        out_type=jax.ShapeDtypeStruct((NN, D), jnp.float32),
        scratch_types=[
            pltpu.VMEM((n_chunk,), jnp.int32),
            pltpu.VMEM((n_chunk, D), jnp.float32),
            pltpu.SemaphoreType.DMA,
        ],
        compiler_params=pltpu.CompilerParams(needs_layout_passes=False),
    )
    def k(z_hbm, emb_hbm, x0_hbm, zc_v, rows_v, sem):
        wid = lax.axis_index("s") * _NC + lax.axis_index("c")
        nbase = jnp.minimum(wid * n_chunk, NN - n_chunk)
        pltpu.sync_copy(z_hbm.at[pl.ds(nbase, n_chunk)], zc_v)
        for t in range(n_chunk // n_sub):
            pltpu.async_copy(
                emb_hbm.at[zc_v.at[pl.ds(t * n_sub, n_sub)]],
                rows_v.at[pl.ds(t * n_sub, n_sub)],
                sem,
            ).wait()
        pltpu.sync_copy(rows_v, x0_hbm.at[pl.ds(nbase, n_chunk)])

    return k(Z, embeddings)


def _tc_body(zi_ref, zj_ref, rbf_ref, emb_ref, wd_ref, wr_ref, br_ref, bd_ref,
             out_ref, tab_ref, wc_ref, bc_ref):
    B = rbf_ref.shape[1]

    @pl.when(pl.program_id(0) == 0)
    def _():
        emb = emb_ref[...]
        wd = wd_ref[...]
        tab_ref[0:128, :] = jnp.dot(
            emb, wd[0:128, :],
            preferred_element_type=jnp.float32).astype(jnp.bfloat16)
        tab_ref[128:256, :] = jnp.dot(
            emb, wd[128:256, :],
            preferred_element_type=jnp.float32).astype(jnp.bfloat16)
        wd3 = wd[256:384, :]
        wc_ref[...] = jnp.dot(wr_ref[...], wd3,
                              preferred_element_type=jnp.float32)
        bc_ref[...] = jnp.dot(br_ref[...], wd3,
                              preferred_element_type=jnp.float32) + bd_ref[...]

    zi = zi_ref[...].reshape(1, B)
    zj = zj_ref[...].reshape(1, B)
    zib = zi.astype(jnp.int16)
    zjb = zj.astype(jnp.int16) + jnp.int16(128)
    iot = lax.broadcasted_iota(jnp.int16, (256, B), 0)
    ones = jnp.ones((256, B), jnp.bfloat16)
    zeros = jnp.zeros((256, B), jnp.bfloat16)
    two_hot = jnp.where((iot == zib) | (iot == zjb), ones, zeros)
    g = lax.dot_general(two_hot, tab_ref[...], (((0,), (0,)), ((), ())),
                        preferred_element_type=jnp.float32)
    p = lax.dot_general(rbf_ref[...], wc_ref[...], (((0,), (0,)), ((), ())),
                        preferred_element_type=jnp.float32)
    out_ref[...] = g + p + bc_ref[...]


def _tc_fused(zi, zj, rbf_t, emb_pad, W_dense, W_rbf, b_rbf, b_dense):
    n_rbf, NE = rbf_t.shape
    B = _EDGE_BLOCK
    NB = NE // B
    zi3 = zi.reshape(NB, 1, B)
    zj3 = zj.reshape(NB, 1, B)
    return pl.pallas_call(
        _tc_body,
        grid=(NB,),
        in_specs=[
            pl.BlockSpec((1, 1, B), lambda i: (i, 0, 0)),
            pl.BlockSpec((1, 1, B), lambda i: (i, 0, 0)),
            pl.BlockSpec((n_rbf, B), lambda i: (0, i)),
            pl.BlockSpec((128, 128), lambda i: (0, 0)),
            pl.BlockSpec((384, 128), lambda i: (0, 0)),
            pl.BlockSpec((n_rbf, 128), lambda i: (0, 0)),
            pl.BlockSpec((1, 128), lambda i: (0, 0)),
            pl.BlockSpec((1, 128), lambda i: (0, 0)),
        ],
        out_specs=pl.BlockSpec((B, 128), lambda i: (i, 0)),
        out_shape=jax.ShapeDtypeStruct((NE, 128), jnp.float32),
        scratch_shapes=[
            pltpu.VMEM((256, 128), jnp.bfloat16),
            pltpu.VMEM((n_rbf, 128), jnp.float32),
            pltpu.VMEM((1, 128), jnp.float32),
        ],
        compiler_params=pltpu.CompilerParams(
            dimension_semantics=("arbitrary",),
        ),
    )(zi3, zj3, rbf_t, emb_pad, W_dense, W_rbf, b_rbf, b_dense)


def kernel(Z, rbf, idx_i, idx_j, embeddings, W_rbf, b_rbf, W_dense, b_dense):
    Z = Z.astype(jnp.int32)
    idx_i = idx_i.astype(jnp.int32)
    idx_j = idx_j.astype(jnp.int32)
    zi, zj = _sc_translate(Z, idx_i, idx_j)
    x0 = _sc_x0(Z, embeddings)
    emb_pad = jnp.pad(embeddings, ((0, 128 - embeddings.shape[0]), (0, 0)))
    x = _tc_fused(zi, zj, rbf.T, emb_pad, W_dense, W_rbf,
                  b_rbf.reshape(1, -1), b_dense.reshape(1, -1))
    return (x, x0)


# speedup vs baseline: 30.3458x; 1.1594x over previous
import functools

import jax
import jax.numpy as jnp
from jax import lax
from jax.experimental import pallas as pl
from jax.experimental.pallas import tpu as pltpu
from jax.experimental.pallas import tpu_sc as plsc

_NC = 2
_NS = 16
_NW = _NC * _NS

_EDGE_BLOCK = 6400


_SC_MESH = plsc.VectorSubcoreMesh(core_axis_name="c", subcore_axis_name="s")


def _sc_translate(Z, idx_i, idx_j):
    NE = idx_i.shape[0]
    NN = Z.shape[0]
    e_per = NE // _NW

    @functools.partial(
        pl.kernel,
        mesh=_SC_MESH,
        out_type=[
            jax.ShapeDtypeStruct((NE,), jnp.int32),
            jax.ShapeDtypeStruct((NE,), jnp.int32),
        ],
        scratch_types=[
            pltpu.VMEM((NN,), jnp.int32),
            pltpu.VMEM((e_per,), jnp.int32),
            pltpu.VMEM((e_per,), jnp.int32),
            pltpu.VMEM((e_per,), jnp.int32),
            pltpu.VMEM((e_per,), jnp.int32),
        ],
        compiler_params=pltpu.CompilerParams(needs_layout_passes=False),
    )
    def k(z_hbm, ii_hbm, jj_hbm, zi_hbm, zj_hbm,
          z_v, ii_v, jj_v, zi_v, zj_v):
        wid = lax.axis_index("s") * _NC + lax.axis_index("c")
        ebase = wid * e_per
        pltpu.sync_copy(z_hbm, z_v)
        pltpu.sync_copy(ii_hbm.at[pl.ds(ebase, e_per)], ii_v)
        pltpu.sync_copy(jj_hbm.at[pl.ds(ebase, e_per)], jj_v)

        @plsc.parallel_loop(0, e_per, step=16, unroll=8)
        def _(t):
            s = pl.ds(t, 16)
            zi_v[s] = plsc.load_gather(z_v, [ii_v[s]])
            zj_v[s] = plsc.load_gather(z_v, [jj_v[s]])

        pltpu.sync_copy(zi_v, zi_hbm.at[pl.ds(ebase, e_per)])
        pltpu.sync_copy(zj_v, zj_hbm.at[pl.ds(ebase, e_per)])

    return k(Z, idx_i, idx_j)


def _sc_x0(Z, embeddings):
    NN = Z.shape[0]
    D = embeddings.shape[1]
    n_chunk = 320
    n_sub = 80

    @functools.partial(
        pl.kernel,
        mesh=_SC_MESH,
        out_type=jax.ShapeDtypeStruct((NN, D), jnp.float32),
        scratch_types=[
            pltpu.VMEM((n_chunk,), jnp.int32),
            pltpu.VMEM((n_chunk, D), jnp.float32),
            pltpu.SemaphoreType.DMA,
        ],
        compiler_params=pltpu.CompilerParams(needs_layout_passes=False),
    )
    def k(z_hbm, emb_hbm, x0_hbm, zc_v, rows_v, sem):
        wid = lax.axis_index("s") * _NC + lax.axis_index("c")
        nbase = jnp.minimum(wid * n_chunk, NN - n_chunk)
        pltpu.sync_copy(z_hbm.at[pl.ds(nbase, n_chunk)], zc_v)
        for t in range(n_chunk // n_sub):
            pltpu.async_copy(
                emb_hbm.at[zc_v.at[pl.ds(t * n_sub, n_sub)]],
                rows_v.at[pl.ds(t * n_sub, n_sub)],
                sem,
            ).wait()
        pltpu.sync_copy(rows_v, x0_hbm.at[pl.ds(nbase, n_chunk)])

    return k(Z, embeddings)


def _tc_body(zi_ref, zj_ref, rbf_ref, emb_ref, wd_ref, wr_ref, br_ref, bd_ref,
             out_ref, tab_ref, wc_ref, bc_ref):
    B = rbf_ref.shape[1]

    @pl.when(pl.program_id(0) == 0)
    def _():
        emb = emb_ref[...]
        wd = wd_ref[...]
        tab_ref[0:128, :] = jnp.dot(
            emb, wd[0:128, :],
            preferred_element_type=jnp.float32).astype(jnp.bfloat16)
        tab_ref[128:256, :] = jnp.dot(
            emb, wd[128:256, :],
            preferred_element_type=jnp.float32).astype(jnp.bfloat16)
        wd3 = wd[256:384, :]
        wc_ref[...] = jnp.dot(wr_ref[...], wd3,
                              preferred_element_type=jnp.float32)
        bc_ref[...] = jnp.dot(br_ref[...], wd3,
                              preferred_element_type=jnp.float32) + bd_ref[...]

    zi = zi_ref[...].reshape(1, B)
    zj = zj_ref[...].reshape(1, B)
    zib = zi.astype(jnp.int16)
    zjb = zj.astype(jnp.int16) + jnp.int16(128)
    iot = lax.broadcasted_iota(jnp.int16, (256, B), 0)
    ones = jnp.ones((256, B), jnp.bfloat16)
    zeros = jnp.zeros((256, B), jnp.bfloat16)
    two_hot = jnp.where((iot == zib) | (iot == zjb), ones, zeros)
    g = lax.dot_general(two_hot, tab_ref[...], (((0,), (0,)), ((), ())),
                        preferred_element_type=jnp.float32)
    p = lax.dot_general(rbf_ref[...], wc_ref[...], (((0,), (0,)), ((), ())),
                        preferred_element_type=jnp.float32)
    out_ref[...] = g + p + bc_ref[...]


def _tc_fused(zi, zj, rbf_t, emb_pad, W_dense, W_rbf, b_rbf, b_dense):
    n_rbf, NE = rbf_t.shape
    B = _EDGE_BLOCK
    NB = NE // B
    zi3 = zi.reshape(NB, 1, B)
    zj3 = zj.reshape(NB, 1, B)
    return pl.pallas_call(
        _tc_body,
        grid=(NB,),
        in_specs=[
            pl.BlockSpec((1, 1, B), lambda i: (i, 0, 0)),
            pl.BlockSpec((1, 1, B), lambda i: (i, 0, 0)),
            pl.BlockSpec((n_rbf, B), lambda i: (0, i)),
            pl.BlockSpec((128, 128), lambda i: (0, 0)),
            pl.BlockSpec((384, 128), lambda i: (0, 0)),
            pl.BlockSpec((n_rbf, 128), lambda i: (0, 0)),
            pl.BlockSpec((1, 128), lambda i: (0, 0)),
            pl.BlockSpec((1, 128), lambda i: (0, 0)),
        ],
        out_specs=pl.BlockSpec((B, 128), lambda i: (i, 0)),
        out_shape=jax.ShapeDtypeStruct((NE, 128), jnp.float32),
        scratch_shapes=[
            pltpu.VMEM((256, 128), jnp.bfloat16),
            pltpu.VMEM((n_rbf, 128), jnp.float32),
            pltpu.VMEM((1, 128), jnp.float32),
        ],
        compiler_params=pltpu.CompilerParams(
            dimension_semantics=("arbitrary",),
        ),
    )(zi3, zj3, rbf_t, emb_pad, W_dense, W_rbf, b_rbf, b_dense)


def kernel(Z, rbf, idx_i, idx_j, embeddings, W_rbf, b_rbf, W_dense, b_dense):
    Z = Z.astype(jnp.int32)
    idx_i = idx_i.astype(jnp.int32)
    idx_j = idx_j.astype(jnp.int32)
    zi, zj = _sc_translate(Z, idx_i, idx_j)
    x0 = _sc_x0(Z, embeddings)
    emb_pad = jnp.pad(embeddings, ((0, 128 - embeddings.shape[0]), (0, 0)))
    x = _tc_fused(zi, zj, rbf.T, emb_pad, W_dense, W_rbf,
                  b_rbf.reshape(1, -1), b_dense.reshape(1, -1))
    return (x, x0)


# speedup vs baseline: 33.6307x; 1.1083x over previous
import functools

import jax
import jax.numpy as jnp
from jax import lax
from jax.experimental import pallas as pl
from jax.experimental.pallas import tpu as pltpu
from jax.experimental.pallas import tpu_sc as plsc

_NC = 2
_NS = 16
_NW = _NC * _NS

_EDGE_BLOCK = 12800


_SC_MESH = plsc.VectorSubcoreMesh(core_axis_name="c", subcore_axis_name="s")


def _sc_translate(Z, idx_i, idx_j):
    NE = idx_i.shape[0]
    NN = Z.shape[0]
    e_per = NE // _NW

    @functools.partial(
        pl.kernel,
        mesh=_SC_MESH,
        out_type=[
            jax.ShapeDtypeStruct((NE,), jnp.int32),
            jax.ShapeDtypeStruct((NE,), jnp.int32),
        ],
        scratch_types=[
            pltpu.VMEM((NN,), jnp.int32),
            pltpu.VMEM((e_per,), jnp.int32),
            pltpu.VMEM((e_per,), jnp.int32),
            pltpu.VMEM((e_per,), jnp.int32),
            pltpu.VMEM((e_per,), jnp.int32),
        ],
        compiler_params=pltpu.CompilerParams(needs_layout_passes=False),
    )
    def k(z_hbm, ii_hbm, jj_hbm, zi_hbm, zj_hbm,
          z_v, ii_v, jj_v, zi_v, zj_v):
        wid = lax.axis_index("s") * _NC + lax.axis_index("c")
        ebase = wid * e_per
        pltpu.sync_copy(z_hbm, z_v)
        pltpu.sync_copy(ii_hbm.at[pl.ds(ebase, e_per)], ii_v)
        pltpu.sync_copy(jj_hbm.at[pl.ds(ebase, e_per)], jj_v)

        @plsc.parallel_loop(0, e_per, step=16, unroll=8)
        def _(t):
            s = pl.ds(t, 16)
            zi_v[s] = plsc.load_gather(z_v, [ii_v[s]])
            zj_v[s] = plsc.load_gather(z_v, [jj_v[s]])

        pltpu.sync_copy(zi_v, zi_hbm.at[pl.ds(ebase, e_per)])
        pltpu.sync_copy(zj_v, zj_hbm.at[pl.ds(ebase, e_per)])

    return k(Z, idx_i, idx_j)


def _sc_x0(Z, embeddings):
    NN = Z.shape[0]
    D = embeddings.shape[1]
    n_chunk = 320
    n_sub = 80

    @functools.partial(
        pl.kernel,
        mesh=_SC_MESH,
        out_type=jax.ShapeDtypeStruct((NN, D), jnp.float32),
        scratch_types=[
            pltpu.VMEM((n_chunk,), jnp.int32),
            pltpu.VMEM((n_chunk, D), jnp.float32),
            pltpu.SemaphoreType.DMA,
        ],
        compiler_params=pltpu.CompilerParams(needs_layout_passes=False),
    )
    def k(z_hbm, emb_hbm, x0_hbm, zc_v, rows_v, sem):
        wid = lax.axis_index("s") * _NC + lax.axis_index("c")
        nbase = jnp.minimum(wid * n_chunk, NN - n_chunk)
        pltpu.sync_copy(z_hbm.at[pl.ds(nbase, n_chunk)], zc_v)
        for t in range(n_chunk // n_sub):
            pltpu.async_copy(
                emb_hbm.at[zc_v.at[pl.ds(t * n_sub, n_sub)]],
                rows_v.at[pl.ds(t * n_sub, n_sub)],
                sem,
            ).wait()
        pltpu.sync_copy(rows_v, x0_hbm.at[pl.ds(nbase, n_chunk)])

    return k(Z, embeddings)


def _tc_body(zi_ref, zj_ref, rbf_ref, emb_ref, wd_ref, wr_ref, br_ref, bd_ref,
             out_ref, tab_ref, wc_ref, bc_ref):
    B = rbf_ref.shape[1]

    @pl.when(pl.program_id(0) == 0)
    def _():
        emb = emb_ref[...]
        wd = wd_ref[...]
        tab_ref[0:128, :] = jnp.dot(
            emb, wd[0:128, :],
            preferred_element_type=jnp.float32).astype(jnp.bfloat16)
        tab_ref[128:256, :] = jnp.dot(
            emb, wd[128:256, :],
            preferred_element_type=jnp.float32).astype(jnp.bfloat16)
        wd3 = wd[256:384, :]
        wc_ref[...] = jnp.dot(wr_ref[...], wd3,
                              preferred_element_type=jnp.float32)
        bc_ref[...] = jnp.dot(br_ref[...], wd3,
                              preferred_element_type=jnp.float32) + bd_ref[...]

    zi = zi_ref[...].reshape(1, B)
    zj = zj_ref[...].reshape(1, B)
    zib = zi.astype(jnp.int16)
    zjb = zj.astype(jnp.int16) + jnp.int16(128)
    iot = lax.broadcasted_iota(jnp.int16, (256, B), 0)
    ones = jnp.ones((256, B), jnp.bfloat16)
    zeros = jnp.zeros((256, B), jnp.bfloat16)
    two_hot = jnp.where((iot == zib) | (iot == zjb), ones, zeros)
    g = lax.dot_general(two_hot, tab_ref[...], (((0,), (0,)), ((), ())),
                        preferred_element_type=jnp.float32)
    p = lax.dot_general(rbf_ref[...], wc_ref[...], (((0,), (0,)), ((), ())),
                        preferred_element_type=jnp.float32)
    out_ref[...] = g + p + bc_ref[...]


def _tc_fused(zi, zj, rbf_t, emb_pad, W_dense, W_rbf, b_rbf, b_dense):
    n_rbf, NE = rbf_t.shape
    B = _EDGE_BLOCK
    NB = NE // B
    zi3 = zi.reshape(NB, 1, B)
    zj3 = zj.reshape(NB, 1, B)
    return pl.pallas_call(
        _tc_body,
        grid=(NB,),
        in_specs=[
            pl.BlockSpec((1, 1, B), lambda i: (i, 0, 0)),
            pl.BlockSpec((1, 1, B), lambda i: (i, 0, 0)),
            pl.BlockSpec((n_rbf, B), lambda i: (0, i)),
            pl.BlockSpec((128, 128), lambda i: (0, 0)),
            pl.BlockSpec((384, 128), lambda i: (0, 0)),
            pl.BlockSpec((n_rbf, 128), lambda i: (0, 0)),
            pl.BlockSpec((1, 128), lambda i: (0, 0)),
            pl.BlockSpec((1, 128), lambda i: (0, 0)),
        ],
        out_specs=pl.BlockSpec((B, 128), lambda i: (i, 0)),
        out_shape=jax.ShapeDtypeStruct((NE, 128), jnp.float32),
        scratch_shapes=[
            pltpu.VMEM((256, 128), jnp.bfloat16),
            pltpu.VMEM((n_rbf, 128), jnp.float32),
            pltpu.VMEM((1, 128), jnp.float32),
        ],
        compiler_params=pltpu.CompilerParams(
            dimension_semantics=("arbitrary",),
        ),
    )(zi3, zj3, rbf_t, emb_pad, W_dense, W_rbf, b_rbf, b_dense)


def kernel(Z, rbf, idx_i, idx_j, embeddings, W_rbf, b_rbf, W_dense, b_dense):
    Z = Z.astype(jnp.int32)
    idx_i = idx_i.astype(jnp.int32)
    idx_j = idx_j.astype(jnp.int32)
    zi, zj = _sc_translate(Z, idx_i, idx_j)
    x0 = _sc_x0(Z, embeddings)
    emb_pad = jnp.pad(embeddings, ((0, 128 - embeddings.shape[0]), (0, 0)))
    x = _tc_fused(zi, zj, rbf.T, emb_pad, W_dense, W_rbf,
                  b_rbf.reshape(1, -1), b_dense.reshape(1, -1))
    return (x, x0)
